# baseline (speedup 1.0000x reference)
import jax
import jax.numpy as jnp
from jax import lax
from jax.experimental import pallas as pl
from jax.experimental.pallas import tpu as pltpu
from jax.experimental.pallas import tpu_sc as plsc

B, N, MO, D = 8, 512, 64, 64
NC, NS = 2, 16
NW = NC * NS
SLOTS = B * N
SLOTS_W = SLOTS // NW
GRP = 8
NGRP = SLOTS_W // GRP
GROWS = GRP * MO
TROWS = SLOTS * MO
ZROW = 2 * TROWS

_MAGIC = 8388608.0


def _sc_body(tab, lc_h, rc_h, sb_h, out_h, cnt_h,
             lc_v, rc_v, sb_v, nc_v, ilist, stg,
             fc_s, td_s, ba_s, bb_s,
             semf, semw0, semw1):
    w = lax.axis_index("s") * NC + lax.axis_index("c")
    slot0 = w * SLOTS_W

    pltpu.sync_copy(lc_h.at[pl.ds(slot0, SLOTS_W)], lc_v)
    pltpu.sync_copy(rc_h.at[pl.ds(slot0, SLOTS_W)], rc_v)
    pltpu.sync_copy(sb_h.at[pl.ds(slot0, SLOTS_W)], sb_v)

    iota16 = lax.broadcasted_iota(jnp.int32, (16,), 0)

    for g in range(SLOTS_W // 16):
        lc16 = lc_v[pl.ds(g * 16, 16)]
        rc16 = rc_v[pl.ds(g * 16, 16)]
        isaft = sb_v[pl.ds(g * 16, 16)] == 1
        fcf = jnp.where(isaft, rc16, lc16)
        scf = jnp.where(isaft, lc16, rc16)
        fc = jnp.clip(((fcf + _MAGIC) - _MAGIC).astype(jnp.int32), 0, MO)
        sc = jnp.clip(((scf + _MAGIC) - _MAGIC).astype(jnp.int32), 0, MO)
        td = fc + jnp.minimum(sc, MO - fc)
        srow = (slot0 + g * 16 + iota16) * MO
        ba = jnp.where(isaft, TROWS, 0) + srow
        bb = jnp.where(isaft, 0, TROWS) + srow
        nc_v[pl.ds(g * 16, 16)] = jnp.minimum(lc16 + rc16, float(MO))
        for lane in range(16):
            li = g * 16 + lane
            fc_s[li] = fc[lane]
            td_s[li] = td[lane]
            ba_s[li] = ba[lane]
            bb_s[li] = bb[lane]
    pltpu.sync_copy(nc_v, cnt_h.at[pl.ds(slot0, SLOTS_W)])

    def group(g, carry):
        par0 = (g & 1) == 0
        pidx = (g & 1) * GROWS
        prow = (g & 1) * GROWS

        @pl.when(jnp.logical_and(g >= 2, par0))
        def _():
            pltpu.make_async_copy(tab.at[pl.ds(0, GROWS)],
                                  stg.at[pl.ds(0, GROWS)], semw0).wait()

        @pl.when(jnp.logical_and(g >= 2, jnp.logical_not(par0)))
        def _():
            pltpu.make_async_copy(tab.at[pl.ds(0, GROWS)],
                                  stg.at[pl.ds(0, GROWS)], semw1).wait()

        for j in range(GRP):
            li = g * GRP + j
            fc = fc_s[li]
            td = td_s[li]
            ba = ba_s[li]
            bb = bb_s[li]
            for k in range(MO // 16):
                mv = iota16 + (k * 16)
                idx = jnp.where(mv < fc, ba + mv,
                                jnp.where(mv < td, bb + (mv - fc), ZROW))
                ilist[pl.ds(pidx + j * MO + k * 16, 16)] = idx

        for k in range(GROWS // 128):
            pltpu.async_copy(
                tab.at[ilist.at[pl.ds(pidx + k * 128, 128)]],
                stg.at[pl.ds(prow + k * 128, 128)], semf)

        pltpu.make_async_copy(tab.at[pl.ds(0, GROWS)],
                              stg.at[pl.ds(0, GROWS)], semf).wait()

        dst_row = pl.multiple_of((slot0 + g * GRP) * MO, GROWS)

        @pl.when(par0)
        def _():
            pltpu.async_copy(stg.at[pl.ds(prow, GROWS)],
                             out_h.at[pl.ds(dst_row, GROWS)], semw0)

        @pl.when(jnp.logical_not(par0))
        def _():
            pltpu.async_copy(stg.at[pl.ds(prow, GROWS)],
                             out_h.at[pl.ds(dst_row, GROWS)], semw1)

        return carry

    lax.fori_loop(0, NGRP, group, jnp.int32(0))

    pltpu.make_async_copy(tab.at[pl.ds(0, GROWS)],
                          stg.at[pl.ds(0, GROWS)], semw0).wait()
    pltpu.make_async_copy(tab.at[pl.ds(0, GROWS)],
                          stg.at[pl.ds(0, GROWS)], semw1).wait()


@jax.jit
def _sc_call(tab, lc, rc, sb):
    mesh = plsc.VectorSubcoreMesh(core_axis_name="c", subcore_axis_name="s")
    return pl.kernel(
        _sc_body,
        out_type=[
            jax.ShapeDtypeStruct((TROWS, D), jnp.float32),
            jax.ShapeDtypeStruct((SLOTS,), jnp.float32),
        ],
        mesh=mesh,
        compiler_params=pltpu.CompilerParams(use_tc_tiling_on_sc=False),
        scratch_types=[
            pltpu.VMEM((SLOTS_W,), jnp.float32),
            pltpu.VMEM((SLOTS_W,), jnp.float32),
            pltpu.VMEM((SLOTS_W,), jnp.int32),
            pltpu.VMEM((SLOTS_W,), jnp.float32),
            pltpu.VMEM((2 * GROWS,), jnp.int32),
            pltpu.VMEM((2 * GROWS, D), jnp.float32),
            pltpu.SMEM((SLOTS_W,), jnp.int32),
            pltpu.SMEM((SLOTS_W,), jnp.int32),
            pltpu.SMEM((SLOTS_W,), jnp.int32),
            pltpu.SMEM((SLOTS_W,), jnp.int32),
            pltpu.SemaphoreType.DMA,
            pltpu.SemaphoreType.DMA,
            pltpu.SemaphoreType.DMA,
        ],
    )(tab, lc, rc, sb)


def kernel(left_buf, left_count, right_buf, right_count, subs):
    tab = jnp.concatenate(
        [left_buf.reshape(TROWS, D), right_buf.reshape(TROWS, D),
         jnp.zeros((8, D), jnp.float32)], axis=0)
    lc = left_count.reshape(SLOTS)
    rc = right_count.reshape(SLOTS)
    sb = subs.reshape(SLOTS)
    out2, out_cnt = _sc_call(tab, lc, rc, sb)
    return out2.reshape(B, N, MO, D), out_cnt.reshape(B, N)


# speedup vs baseline: 7.9108x; 7.9108x over previous
import jax
import jax.numpy as jnp
from jax import lax
from jax.experimental import pallas as pl
from jax.experimental.pallas import tpu as pltpu
from jax.experimental.pallas import tpu_sc as plsc

B, N, MO, D = 8, 512, 64, 64
NC, NS = 2, 16
NW = NC * NS
SLOTS = B * N
SLOTS_W = SLOTS // NW
SLOT_EL = MO * D
GRP = 8
GRP_EL = GRP * SLOT_EL

_MAGIC = 8388608.0


def _copy_tree(length, lo, hi, emit):
    if lo == hi:
        if lo > 0:
            emit(lo)
        return
    mid = (lo + hi) // 2

    @pl.when(length <= mid)
    def _():
        _copy_tree(length, lo, mid, emit)

    @pl.when(length > mid)
    def _():
        _copy_tree(length, mid + 1, hi, emit)


def _sc_body(lt, rt, lc_h, rc_h, sb_h, out_h, cnt_h,
             lc_v, rc_v, sb_v, nc_v, stg,
             lenl_s, dstl_s, lenr_s, dstr_s,
             semf, semw0, semw1):
    w = lax.axis_index("s") * NC + lax.axis_index("c")
    slot0 = w * SLOTS_W

    pltpu.sync_copy(lc_h.at[pl.ds(slot0, SLOTS_W)], lc_v)
    pltpu.sync_copy(rc_h.at[pl.ds(slot0, SLOTS_W)], rc_v)
    pltpu.sync_copy(sb_h.at[pl.ds(slot0, SLOTS_W)], sb_v)

    for g in range(SLOTS_W // 16):
        lc16 = lc_v[pl.ds(g * 16, 16)]
        rc16 = rc_v[pl.ds(g * 16, 16)]
        isaft = sb_v[pl.ds(g * 16, 16)] == 1
        fcf = jnp.where(isaft, rc16, lc16)
        scf = jnp.where(isaft, lc16, rc16)
        fc = jnp.clip(((fcf + _MAGIC) - _MAGIC).astype(jnp.int32), 0, MO)
        sc = jnp.clip(((scf + _MAGIC) - _MAGIC).astype(jnp.int32), 0, MO)
        sc = jnp.minimum(sc, MO - fc)
        len_l = jnp.where(isaft, sc, fc)
        dst_l = jnp.where(isaft, fc, 0)
        len_r = jnp.where(isaft, fc, sc)
        dst_r = jnp.where(isaft, 0, fc)
        nc_v[pl.ds(g * 16, 16)] = jnp.minimum(lc16 + rc16, float(MO))
        for lane in range(16):
            li = g * 16 + lane
            lenl_s[li] = len_l[lane]
            dstl_s[li] = dst_l[lane]
            lenr_s[li] = len_r[lane]
            dstr_s[li] = dst_r[lane]
    pltpu.sync_copy(nc_v, cnt_h.at[pl.ds(slot0, SLOTS_W)])

    z16 = jnp.zeros((16,), jnp.float32)

    def slot_step(s, rows_acc):
        j = s & 7
        g = lax.shift_right_logical(s, 3)
        par0 = (g & 1) == 0
        pbase = (g & 1) * GRP_EL
        src_el = (slot0 + s) * SLOT_EL
        sb_el = pbase + j * SLOT_EL

        @pl.when(jnp.logical_and(j == 0, jnp.logical_and(g >= 2, par0)))
        def _():
            pltpu.make_async_copy(lt.at[pl.ds(0, GRP_EL)],
                                  stg.at[pl.ds(0, GRP_EL)], semw0).wait()

        @pl.when(jnp.logical_and(
            j == 0, jnp.logical_and(g >= 2, jnp.logical_not(par0))))
        def _():
            pltpu.make_async_copy(lt.at[pl.ds(0, GRP_EL)],
                                  stg.at[pl.ds(0, GRP_EL)], semw1).wait()

        len_l = lenl_s[s]
        dst_l = dstl_s[s]
        len_r = lenr_s[s]
        dst_r = dstr_s[s]
        td = len_l + len_r

        def emit_l(rows):
            so = pl.multiple_of(src_el, D)
            do = pl.multiple_of(sb_el + dst_l * D, D)
            pltpu.async_copy(lt.at[pl.ds(so, rows * D)],
                             stg.at[pl.ds(do, rows * D)], semf)

        def emit_r(rows):
            so = pl.multiple_of(src_el, D)
            do = pl.multiple_of(sb_el + dst_r * D, D)
            pltpu.async_copy(rt.at[pl.ds(so, rows * D)],
                             stg.at[pl.ds(do, rows * D)], semf)

        _copy_tree(len_l, 0, MO, emit_l)
        _copy_tree(len_r, 0, MO, emit_r)

        def zrow(r, c):
            base = sb_el + r * D
            for k in range(D // 16):
                stg[pl.ds(base + k * 16, 16)] = z16
            return c

        lax.fori_loop(td, MO, zrow, jnp.int32(0))

        rows_new = rows_acc + td

        @pl.when(j == 7)
        def _():
            for bit in (512, 256, 128, 64, 32, 16, 8, 4, 2, 1):
                @pl.when((rows_new & bit) != 0)
                def _(bit=bit):
                    pltpu.make_async_copy(
                        lt.at[pl.ds(0, bit * D)],
                        stg.at[pl.ds(0, bit * D)], semf).wait()

            dst_el = pl.multiple_of((slot0 + (s & ~7)) * SLOT_EL, D)

            @pl.when(par0)
            def _():
                pltpu.async_copy(stg.at[pl.ds(pbase, GRP_EL)],
                                 out_h.at[pl.ds(dst_el, GRP_EL)], semw0)

            @pl.when(jnp.logical_not(par0))
            def _():
                pltpu.async_copy(stg.at[pl.ds(pbase, GRP_EL)],
                                 out_h.at[pl.ds(dst_el, GRP_EL)], semw1)

        return jnp.where(j == 7, 0, rows_new)

    lax.fori_loop(0, SLOTS_W, slot_step, jnp.int32(0))

    pltpu.make_async_copy(lt.at[pl.ds(0, GRP_EL)],
                          stg.at[pl.ds(0, GRP_EL)], semw0).wait()
    pltpu.make_async_copy(lt.at[pl.ds(0, GRP_EL)],
                          stg.at[pl.ds(0, GRP_EL)], semw1).wait()


@jax.jit
def _sc_call(lt, rt, lc, rc, sb):
    mesh = plsc.VectorSubcoreMesh(core_axis_name="c", subcore_axis_name="s")
    return pl.kernel(
        _sc_body,
        out_type=[
            jax.ShapeDtypeStruct((SLOTS * SLOT_EL,), jnp.float32),
            jax.ShapeDtypeStruct((SLOTS,), jnp.float32),
        ],
        mesh=mesh,
        compiler_params=pltpu.CompilerParams(use_tc_tiling_on_sc=False),
        scratch_types=[
            pltpu.VMEM((SLOTS_W,), jnp.float32),
            pltpu.VMEM((SLOTS_W,), jnp.float32),
            pltpu.VMEM((SLOTS_W,), jnp.int32),
            pltpu.VMEM((SLOTS_W,), jnp.float32),
            pltpu.VMEM((2 * GRP_EL,), jnp.float32),
            pltpu.SMEM((SLOTS_W,), jnp.int32),
            pltpu.SMEM((SLOTS_W,), jnp.int32),
            pltpu.SMEM((SLOTS_W,), jnp.int32),
            pltpu.SMEM((SLOTS_W,), jnp.int32),
            pltpu.SemaphoreType.DMA,
            pltpu.SemaphoreType.DMA,
            pltpu.SemaphoreType.DMA,
        ],
    )(lt, rt, lc, rc, sb)


def kernel(left_buf, left_count, right_buf, right_count, subs):
    lt = left_buf.reshape(SLOTS * SLOT_EL)
    rt = right_buf.reshape(SLOTS * SLOT_EL)
    lc = left_count.reshape(SLOTS)
    rc = right_count.reshape(SLOTS)
    sb = subs.reshape(SLOTS)
    out_flat, out_cnt = _sc_call(lt, rt, lc, rc, sb)
    return out_flat.reshape(B, N, MO, D), out_cnt.reshape(B, N)


# speedup vs baseline: 7.9113x; 1.0001x over previous
import jax
import jax.numpy as jnp
from jax import lax
from jax.experimental import pallas as pl
from jax.experimental.pallas import tpu as pltpu
from jax.experimental.pallas import tpu_sc as plsc

B, N, MO, D = 8, 512, 64, 64
NC, NS = 2, 16
NW = NC * NS
SLOTS = B * N
SLOTS_W = SLOTS // NW
WPB = N // SLOTS_W
GRP = 8
NBUF = 2 * GRP

_MAGIC = 8388608.0


def _copy_tree(length, lo, hi, emit):
    if lo == hi:
        if lo > 0:
            emit(lo)
        return
    mid = (lo + hi) // 2

    @pl.when(length <= mid)
    def _():
        _copy_tree(length, lo, mid, emit)

    @pl.when(length > mid)
    def _():
        _copy_tree(length, mid + 1, hi, emit)


def _sc_body(lt, rt, lc_h, rc_h, sb_h, out_h, cnt_h,
             lc_v, rc_v, sb_v, nc_v, stg,
             lenl_s, dstl_s, lenr_s, dstr_s,
             semf, semw0, semw1):
    w = lax.axis_index("s") * NC + lax.axis_index("c")
    bi = lax.shift_right_logical(w, 2)
    n0 = (w & (WPB - 1)) * SLOTS_W

    pltpu.sync_copy(lc_h.at[bi, pl.ds(n0, SLOTS_W)], lc_v)
    pltpu.sync_copy(rc_h.at[bi, pl.ds(n0, SLOTS_W)], rc_v)
    pltpu.sync_copy(sb_h.at[bi, pl.ds(n0, SLOTS_W)], sb_v)

    for g in range(SLOTS_W // 16):
        lc16 = lc_v[pl.ds(g * 16, 16)]
        rc16 = rc_v[pl.ds(g * 16, 16)]
        isaft = sb_v[pl.ds(g * 16, 16)] == 1
        fcf = jnp.where(isaft, rc16, lc16)
        scf = jnp.where(isaft, lc16, rc16)
        fc = jnp.clip(((fcf + _MAGIC) - _MAGIC).astype(jnp.int32), 0, MO)
        sc = jnp.clip(((scf + _MAGIC) - _MAGIC).astype(jnp.int32), 0, MO)
        sc = jnp.minimum(sc, MO - fc)
        len_l = jnp.where(isaft, sc, fc)
        dst_l = jnp.where(isaft, fc, 0)
        len_r = jnp.where(isaft, fc, sc)
        dst_r = jnp.where(isaft, 0, fc)
        nc_v[pl.ds(g * 16, 16)] = jnp.minimum(lc16 + rc16, float(MO))
        for lane in range(16):
            li = g * 16 + lane
            lenl_s[li] = len_l[lane]
            dstl_s[li] = dst_l[lane]
            lenr_s[li] = len_r[lane]
            dstr_s[li] = dst_r[lane]
    pltpu.sync_copy(nc_v, cnt_h.at[bi, pl.ds(n0, SLOTS_W)])

    z16 = jnp.zeros((16,), jnp.float32)

    def slot_step(s, rows_acc):
        j = s & 7
        g = lax.shift_right_logical(s, 3)
        par0 = (g & 1) == 0
        bufs = (g & 1) * GRP + j
        n = n0 + s

        @pl.when(jnp.logical_and(j == 0, jnp.logical_and(g >= 2, par0)))
        def _():
            pltpu.make_async_copy(lt.at[bi, pl.ds(n0, GRP)],
                                  stg.at[pl.ds(0, GRP)], semw0).wait()

        @pl.when(jnp.logical_and(
            j == 0, jnp.logical_and(g >= 2, jnp.logical_not(par0))))
        def _():
            pltpu.make_async_copy(lt.at[bi, pl.ds(n0, GRP)],
                                  stg.at[pl.ds(0, GRP)], semw1).wait()

        len_l = lenl_s[s]
        dst_l = dstl_s[s]
        len_r = lenr_s[s]
        dst_r = dstr_s[s]
        td = len_l + len_r

        def emit_l(rows):
            pltpu.async_copy(lt.at[bi, n, pl.ds(0, rows), :],
                             stg.at[bufs, pl.ds(dst_l, rows), :], semf)

        def emit_r(rows):
            pltpu.async_copy(rt.at[bi, n, pl.ds(0, rows), :],
                             stg.at[bufs, pl.ds(dst_r, rows), :], semf)

        _copy_tree(len_l, 0, MO, emit_l)
        _copy_tree(len_r, 0, MO, emit_r)

        def zrow(r, c):
            for k in range(D // 16):
                stg[bufs, r, pl.ds(k * 16, 16)] = z16
            return c

        lax.fori_loop(td, MO, zrow, jnp.int32(0))

        rows_new = rows_acc + td

        @pl.when(j == 7)
        def _():
            for bit in (8, 4, 2, 1):
                @pl.when((rows_new & (bit * MO)) != 0)
                def _(bit=bit):
                    pltpu.make_async_copy(
                        lt.at[bi, pl.ds(n0, bit)],
                        stg.at[pl.ds(0, bit)], semf).wait()
            for bit in (32, 16, 8, 4, 2, 1):
                @pl.when((rows_new & bit) != 0)
                def _(bit=bit):
                    pltpu.make_async_copy(
                        lt.at[bi, n0, pl.ds(0, bit), :],
                        stg.at[0, pl.ds(0, bit), :], semf).wait()

            gbase = n0 + (s & ~7)
            pb = (g & 1) * GRP

            @pl.when(par0)
            def _():
                pltpu.async_copy(stg.at[pl.ds(pb, GRP)],
                                 out_h.at[bi, pl.ds(gbase, GRP)], semw0)

            @pl.when(jnp.logical_not(par0))
            def _():
                pltpu.async_copy(stg.at[pl.ds(pb, GRP)],
                                 out_h.at[bi, pl.ds(gbase, GRP)], semw1)

        return jnp.where(j == 7, 0, rows_new)

    lax.fori_loop(0, SLOTS_W, slot_step, jnp.int32(0))

    pltpu.make_async_copy(lt.at[bi, pl.ds(n0, GRP)],
                          stg.at[pl.ds(0, GRP)], semw0).wait()
    pltpu.make_async_copy(lt.at[bi, pl.ds(n0, GRP)],
                          stg.at[pl.ds(0, GRP)], semw1).wait()


@jax.jit
def _sc_call(lt, rt, lc, rc, sb):
    mesh = plsc.VectorSubcoreMesh(core_axis_name="c", subcore_axis_name="s")
    return pl.kernel(
        _sc_body,
        out_type=[
            jax.ShapeDtypeStruct((B, N, MO, D), jnp.float32),
            jax.ShapeDtypeStruct((B, N), jnp.float32),
        ],
        mesh=mesh,
        compiler_params=pltpu.CompilerParams(use_tc_tiling_on_sc=False),
        scratch_types=[
            pltpu.VMEM((SLOTS_W,), jnp.float32),
            pltpu.VMEM((SLOTS_W,), jnp.float32),
            pltpu.VMEM((SLOTS_W,), jnp.int32),
            pltpu.VMEM((SLOTS_W,), jnp.float32),
            pltpu.VMEM((NBUF, MO, D), jnp.float32),
            pltpu.SMEM((SLOTS_W,), jnp.int32),
            pltpu.SMEM((SLOTS_W,), jnp.int32),
            pltpu.SMEM((SLOTS_W,), jnp.int32),
            pltpu.SMEM((SLOTS_W,), jnp.int32),
            pltpu.SemaphoreType.DMA,
            pltpu.SemaphoreType.DMA,
            pltpu.SemaphoreType.DMA,
        ],
    )(lt, rt, lc, rc, sb)


def kernel(left_buf, left_count, right_buf, right_count, subs):
    out, cnt = _sc_call(left_buf, right_buf, left_count, right_count, subs)
    return out, cnt
